# Initial kernel scaffold; baseline (speedup 1.0000x reference)
import jax
import jax.numpy as jnp
from jax.experimental import pallas as pl


def kernel(x, table):
    raise NotImplementedError("write your pallas kernel here")


# speedup vs baseline: 1.4600x; 1.4600x over previous
import functools

import jax
import jax.numpy as jnp
from jax import lax
from jax.experimental import pallas as pl
from jax.experimental.pallas import tpu as pltpu
from jax.experimental.pallas import tpu_sc as plsc

NUM_CORES = 2
NUM_SUBCORES = 16
NUM_WORKERS = NUM_CORES * NUM_SUBCORES
CHUNK = 1024


@jax.jit
def _embed(xf, table):
    (n,) = xf.shape
    _, d = table.shape
    n_per_w = n // NUM_WORKERS
    n_chunks = n_per_w // CHUNK
    mesh = plsc.VectorSubcoreMesh(
        core_axis_name="c", subcore_axis_name="s"
    )

    @functools.partial(
        pl.kernel,
        out_type=jax.ShapeDtypeStruct((n, d), jnp.float32),
        mesh=mesh,
        scratch_types=[
            pltpu.VMEM((CHUNK,), jnp.int32),
            pltpu.VMEM((CHUNK, d), jnp.float32),
            pltpu.SemaphoreType.DMA,
        ],
        compiler_params=pltpu.CompilerParams(use_tc_tiling_on_sc=False),
    )
    def emb(x_hbm, tab_hbm, out_hbm, idx_v, rows_v, sem):
        wid = lax.axis_index("s") * NUM_CORES + lax.axis_index("c")
        base = wid * n_per_w

        def body(i, carry):
            off = base + i * CHUNK
            pltpu.sync_copy(x_hbm.at[pl.ds(off, CHUNK)], idx_v)
            pltpu.async_copy(tab_hbm.at[idx_v], rows_v, sem).wait()
            pltpu.sync_copy(rows_v, out_hbm.at[pl.ds(off, CHUNK)])
            return carry

        lax.fori_loop(0, n_chunks, body, 0)

    return emb(xf, table)


def kernel(x, table):
    b, l = x.shape
    _, d = table.shape
    out = _embed(x.reshape(b * l), table)
    return out.reshape(b, l, d)


# speedup vs baseline: 1.5035x; 1.0298x over previous
import functools

import jax
import jax.numpy as jnp
from jax import lax
from jax.experimental import pallas as pl
from jax.experimental.pallas import tpu as pltpu
from jax.experimental.pallas import tpu_sc as plsc

NUM_CORES = 2
NUM_SUBCORES = 16
NUM_WORKERS = NUM_CORES * NUM_SUBCORES
CHUNK = 1280


@jax.jit
def _embed(xf, table):
    (n,) = xf.shape
    _, d = table.shape
    n_per_w = n // NUM_WORKERS
    n_chunks = n_per_w // CHUNK
    assert n_chunks % 2 == 0 and n_chunks >= 4
    mesh = plsc.VectorSubcoreMesh(core_axis_name="c", subcore_axis_name="s")

    @functools.partial(
        pl.kernel,
        out_type=jax.ShapeDtypeStruct((n, d), jnp.float32),
        mesh=mesh,
        scratch_types=[
            pltpu.VMEM((n_per_w,), jnp.int32),
            pltpu.VMEM((CHUNK, d), jnp.float32),
            pltpu.VMEM((CHUNK, d), jnp.float32),
            pltpu.SemaphoreType.DMA,
            pltpu.SemaphoreType.DMA,
            pltpu.SemaphoreType.DMA,
            pltpu.SemaphoreType.DMA,
        ],
        compiler_params=pltpu.CompilerParams(use_tc_tiling_on_sc=False),
    )
    def emb(x_hbm, tab_hbm, out_hbm, idx_v, rows0, rows1, gs0, gs1, ws0, ws1):
        wid = lax.axis_index("s") * NUM_CORES + lax.axis_index("c")
        base = wid * n_per_w
        rows = (rows0, rows1)
        gsem = (gs0, gs1)
        wsem = (ws0, ws1)

        pltpu.sync_copy(x_hbm.at[pl.ds(base, n_per_w)], idx_v)

        def start_gather(g, b):
            pltpu.async_copy(
                tab_hbm.at[idx_v.at[pl.ds(g * CHUNK, CHUNK)]], rows[b], gsem[b]
            )

        def wait_gather(g, b):
            pltpu.make_async_copy(
                tab_hbm.at[idx_v.at[pl.ds(g * CHUNK, CHUNK)]], rows[b], gsem[b]
            ).wait()

        def start_wb(g, b):
            pltpu.async_copy(
                rows[b], out_hbm.at[pl.ds(base + g * CHUNK, CHUNK)], wsem[b]
            )

        def wait_wb(g, b):
            pltpu.make_async_copy(
                rows[b], out_hbm.at[pl.ds(base + g * CHUNK, CHUNK)], wsem[b]
            ).wait()

        start_gather(0, 0)
        start_gather(1, 1)

        def pair_body(p, carry):
            for b in (0, 1):
                g = 2 * p + b
                wait_gather(g, b)
                start_wb(g, b)
                wait_wb(g, b)
                start_gather(g + 2, b)
            return carry

        lax.fori_loop(0, (n_chunks - 2) // 2, pair_body, 0)

        for b in (0, 1):
            g = n_chunks - 2 + b
            wait_gather(g, b)
            start_wb(g, b)
            wait_wb(g, b)

    return emb(xf, table)


def kernel(x, table):
    b, l = x.shape
    _, d = table.shape
    out = _embed(x.reshape(b * l), table)
    return out.reshape(b, l, d)
